# baseline (speedup 1.0000x reference)
import functools

import jax
import jax.numpy as jnp
from jax import lax
from jax.experimental import pallas as pl
from jax.experimental.pallas import tpu as pltpu
from jax.experimental.pallas import tpu_sc as plsc

EMBED_DIM = 32
BATCH = 16384
GROUP = 4
NUM_CORES = 2
NUM_SUBCORES = 16
NUM_WORKERS = NUM_CORES * NUM_SUBCORES
B_PER_W = BATCH // NUM_WORKERS
CHUNK = 128
N_CHUNKS = B_PER_W // CHUNK
LANE = 16


def _gather_body(uidx_hbm, midx_hbm, utab_hbm, mtab_hbm, uout_hbm, mout_hbm,
                 idx_v, grp_v, rows_v, sem):
    wid = lax.axis_index("s") * NUM_CORES + lax.axis_index("c")
    base = wid * B_PER_W

    def one_table(idx_hbm, tab_hbm, out_hbm):
        pltpu.sync_copy(idx_hbm.at[pl.ds(base, B_PER_W)], idx_v)
        for v in range(B_PER_W // LANE):
            s = v * LANE
            grp_v[pl.ds(s, LANE)] = lax.shift_right_logical(
                idx_v[pl.ds(s, LANE)], 2)
        for j in range(N_CHUNKS):
            s = j * CHUNK
            pltpu.async_copy(tab_hbm.at[grp_v.at[pl.ds(s, CHUNK)]],
                             rows_v.at[pl.ds(s, CHUNK)], sem)
        for j in range(N_CHUNKS):
            s = j * CHUNK
            pltpu.make_async_copy(tab_hbm.at[grp_v.at[pl.ds(s, CHUNK)]],
                                  rows_v.at[pl.ds(s, CHUNK)], sem).wait()
        pltpu.sync_copy(rows_v, out_hbm.at[pl.ds(base, B_PER_W)])

    one_table(uidx_hbm, utab_hbm, uout_hbm)
    one_table(midx_hbm, mtab_hbm, mout_hbm)


@jax.jit
def _sc_gather(user_ids, movie_ids, utab128, mtab128):
    mesh = plsc.VectorSubcoreMesh(core_axis_name="c", subcore_axis_name="s")
    fn = functools.partial(
        pl.kernel,
        mesh=mesh,
        compiler_params=pltpu.CompilerParams(use_tc_tiling_on_sc=True),
        out_type=[
            jax.ShapeDtypeStruct((BATCH, GROUP * EMBED_DIM), jnp.float32),
            jax.ShapeDtypeStruct((BATCH, GROUP * EMBED_DIM), jnp.float32),
        ],
        scratch_types=[
            pltpu.VMEM((B_PER_W,), jnp.int32),
            pltpu.VMEM((B_PER_W,), jnp.int32),
            pltpu.VMEM((B_PER_W, GROUP * EMBED_DIM), jnp.float32),
            pltpu.SemaphoreType.DMA,
        ],
    )(_gather_body)
    return fn(user_ids, movie_ids, utab128, mtab128)


def _mlp_body(u_ref, m_ref, uoff_ref, moff_ref, w1a_ref, w1b_ref, b1_ref,
              w2_ref, b2_ref, w3_ref, b3_ref, out_ref):
    def extract(rows, off):
        sel = off[:, None]
        x = jnp.where(sel == 0, rows[:, 0 * EMBED_DIM:1 * EMBED_DIM], 0.0)
        x = x + jnp.where(sel == 1, rows[:, 1 * EMBED_DIM:2 * EMBED_DIM], 0.0)
        x = x + jnp.where(sel == 2, rows[:, 2 * EMBED_DIM:3 * EMBED_DIM], 0.0)
        x = x + jnp.where(sel == 3, rows[:, 3 * EMBED_DIM:4 * EMBED_DIM], 0.0)
        return x

    u = extract(u_ref[...], uoff_ref[...])
    m = extract(m_ref[...], moff_ref[...])
    h = jnp.dot(u, w1a_ref[...], preferred_element_type=jnp.float32)
    h = h + jnp.dot(m, w1b_ref[...], preferred_element_type=jnp.float32)
    h = jnp.maximum(h + b1_ref[...], 0.0)
    h2 = jnp.dot(h, w2_ref[...], preferred_element_type=jnp.float32)
    h2 = jnp.maximum(h2 + b2_ref[...], 0.0)
    out_ref[...] = jnp.sum(h2 * w3_ref[...], axis=1) + b3_ref[0, 0]


def _tc_mlp(u, m, uoff, moff, W1, b1, W2, b2, W3, b3, bm=2048):
    w1a = W1[:EMBED_DIM]
    w1b = W1[EMBED_DIM:]
    b1r = b1.reshape(1, -1)
    b2r = b2.reshape(1, -1)
    w3r = W3.reshape(1, -1)
    b3r = b3.reshape(1, 1)
    grid = (BATCH // bm,)
    wide = GROUP * EMBED_DIM
    return pl.pallas_call(
        _mlp_body,
        grid=grid,
        in_specs=[
            pl.BlockSpec((bm, wide), lambda i: (i, 0)),
            pl.BlockSpec((bm, wide), lambda i: (i, 0)),
            pl.BlockSpec((bm,), lambda i: (i,)),
            pl.BlockSpec((bm,), lambda i: (i,)),
            pl.BlockSpec(w1a.shape, lambda i: (0, 0)),
            pl.BlockSpec(w1b.shape, lambda i: (0, 0)),
            pl.BlockSpec(b1r.shape, lambda i: (0, 0)),
            pl.BlockSpec(W2.shape, lambda i: (0, 0)),
            pl.BlockSpec(b2r.shape, lambda i: (0, 0)),
            pl.BlockSpec(w3r.shape, lambda i: (0, 0)),
            pl.BlockSpec(b3r.shape, lambda i: (0, 0)),
        ],
        out_specs=pl.BlockSpec((bm,), lambda i: (i,)),
        out_shape=jax.ShapeDtypeStruct((BATCH,), jnp.float32),
    )(u, m, uoff, moff, w1a, w1b, b1r, W2, b2r, w3r, b3r)


def kernel(user_ids, movie_ids, user_table, movie_table, W1, b1, W2, b2, W3, b3):
    uids = user_ids.astype(jnp.int32)
    mids = movie_ids.astype(jnp.int32)
    utab128 = user_table.reshape(-1, GROUP * EMBED_DIM)
    mtab128 = movie_table.reshape(-1, GROUP * EMBED_DIM)
    u, m = _sc_gather(uids, mids, utab128, mtab128)
    return _tc_mlp(u, m, uids & 3, mids & 3, W1, b1, W2, b2, W3, b3)
